# baseline (speedup 1.0000x reference)
import jax
import jax.numpy as jnp
from jax.experimental import pallas as pl
from jax.experimental.pallas import tpu as pltpu

_BM = 80


def _fused(feat_ref, w_ref, adj_ref, out_ref, y_ref):
    @pl.when(pl.program_id(0) == 0)
    def _():
        y_ref[...] = jnp.dot(
            feat_ref[...], w_ref[...], preferred_element_type=jnp.float32
        ).astype(jnp.bfloat16)

    out_ref[...] = jnp.dot(
        adj_ref[...].astype(jnp.bfloat16),
        y_ref[...],
        preferred_element_type=jnp.float32,
    )


def kernel(feat, adj, weight):
    n, in_feat = feat.shape
    out_feat = weight.shape[1]
    bm = _BM if n % _BM == 0 else n
    return pl.pallas_call(
        _fused,
        grid=(n // bm,),
        in_specs=[
            pl.BlockSpec((n, in_feat), lambda i: (0, 0)),
            pl.BlockSpec((in_feat, out_feat), lambda i: (0, 0)),
            pl.BlockSpec((bm, n), lambda i: (i, 0)),
        ],
        out_specs=pl.BlockSpec((bm, out_feat), lambda i: (i, 0)),
        out_shape=jax.ShapeDtypeStruct((n, out_feat), jnp.float32),
        scratch_shapes=[pltpu.VMEM((n, out_feat), jnp.bfloat16)],
    )(feat, weight, adj)


# speedup vs baseline: 1.3896x; 1.3896x over previous
import jax
import jax.numpy as jnp
from jax.experimental import pallas as pl
from jax.experimental.pallas import tpu as pltpu

_BM = 400


def _fused(feat_ref, w_ref, adj_ref, out_ref, y_ref):
    @pl.when(pl.program_id(0) == 0)
    def _():
        y_ref[...] = jnp.dot(
            feat_ref[...], w_ref[...], preferred_element_type=jnp.float32
        ).astype(jnp.bfloat16)

    out_ref[...] = jnp.dot(
        adj_ref[...].astype(jnp.bfloat16),
        y_ref[...],
        preferred_element_type=jnp.float32,
    )


def kernel(feat, adj, weight):
    n, in_feat = feat.shape
    out_feat = weight.shape[1]
    bm = _BM if n % _BM == 0 else n
    return pl.pallas_call(
        _fused,
        grid=(n // bm,),
        in_specs=[
            pl.BlockSpec((n, in_feat), lambda i: (0, 0)),
            pl.BlockSpec((in_feat, out_feat), lambda i: (0, 0)),
            pl.BlockSpec((bm, n), lambda i: (i, 0)),
        ],
        out_specs=pl.BlockSpec((bm, out_feat), lambda i: (i, 0)),
        out_shape=jax.ShapeDtypeStruct((n, out_feat), jnp.float32),
        scratch_shapes=[pltpu.VMEM((n, out_feat), jnp.bfloat16)],
    )(feat, weight, adj)
